# Initial kernel scaffold; baseline (speedup 1.0000x reference)
import jax
import jax.numpy as jnp
from jax.experimental import pallas as pl


def kernel(x, edge_index, W1, b1, W2, b2, W3, b3, W4, b4, W5, b5):
    raise NotImplementedError("write your pallas kernel here")


# speedup vs baseline: 9.0294x; 9.0294x over previous
import functools

import jax
import jax.numpy as jnp
from jax import lax
from jax.experimental import pallas as pl
from jax.experimental.pallas import tpu as pltpu
from jax.experimental.pallas import tpu_sc as plsc

NC = 2
NS = 16
NW = NC * NS
SUB = 128
GB = 1024
BR = 512


@functools.lru_cache(maxsize=None)
def _make_prop(n_pad: int, C: int, ng: int, gb: int):
    NSUB = gb // SUB
    RW = n_pad // NS
    WO = RW // 4
    mesh = plsc.VectorSubcoreMesh(
        core_axis_name="c", subcore_axis_name="s",
        num_cores=NC, num_subcores=NS)

    @functools.partial(
        pl.kernel,
        out_type=jax.ShapeDtypeStruct((NC, n_pad, C), jnp.float32),
        mesh=mesh,
        scratch_types=[
            pltpu.VMEM((NSUB, SUB), jnp.int32),
            pltpu.VMEM((NSUB, SUB), jnp.int32),
            pltpu.VMEM((gb, C), jnp.float32),
            pltpu.VMEM_SHARED((n_pad, C), jnp.float32),
            pltpu.SemaphoreType.DMA,
        ],
        compiler_params=pltpu.CompilerParams(use_tc_tiling_on_sc=False),
    )
    def prop(y_hbm, gidx_hbm, sidx_hbm, zeros_hbm, out_hbm,
             gv, sv, rows, acc, sem):
        cid = lax.axis_index("c")
        sid = lax.axis_index("s")
        tid = cid * NS + sid
        pltpu.sync_copy(zeros_hbm, acc.at[pl.ds(sid * RW, RW)])
        plsc.subcore_barrier()

        def body(g, _):
            grp = tid * ng + g
            pltpu.sync_copy(gidx_hbm.at[grp], gv)
            pltpu.sync_copy(sidx_hbm.at[grp], sv)
            descs = [
                pltpu.async_copy(y_hbm.at[gv.at[j]],
                                 rows.at[pl.ds(j * SUB, SUB)], sem)
                for j in range(NSUB)
            ]
            for d in descs:
                d.wait()
            for j in range(NSUB):
                pltpu.sync_copy(rows.at[pl.ds(j * SUB, SUB)],
                                acc.at[sv.at[j]], add=True)
            return 0

        lax.fori_loop(0, ng, body, 0)
        plsc.subcore_barrier()
        for w in range(4):
            row0 = sid * RW + w * WO
            pltpu.sync_copy(acc.at[pl.ds(row0, WO)],
                            out_hbm.at[cid, pl.ds(row0, WO)])

    return prop


def _row_spec(C):
    return pl.BlockSpec((BR, C), lambda i: (i, 0))


def _full_spec(R, C):
    return pl.BlockSpec((R, C), lambda i: (0, 0))


def _tc_call(body, n_pad, in_specs, out_cs, args):
    return pl.pallas_call(
        body,
        grid=(n_pad // BR,),
        in_specs=in_specs,
        out_specs=[_row_spec(c) for c in out_cs],
        out_shape=[jax.ShapeDtypeStruct((n_pad, c), jnp.float32)
                   for c in out_cs],
    )(*args)


def _head_deg(x_pad, wcat, deg0, deg1):
    n_pad, F = x_pad.shape

    def body(x_ref, w_ref, d0_ref, d1_ref, a_ref, y_ref, dv_ref):
        deg = d0_ref[:, 0] + d1_ref[:, 0]
        d = jnp.where(deg > 0, lax.rsqrt(deg), 0.0)
        h = jnp.dot(x_ref[:], w_ref[:], preferred_element_type=jnp.float32)
        a_ref[:] = h[:, :16]
        y_ref[:] = h[:, 16:48] * d[:, None]
        dv_ref[:] = jnp.broadcast_to(d[:, None], (BR, 8))

    return _tc_call(body, n_pad,
                    [_row_spec(F), _full_spec(F, 48), _row_spec(8), _row_spec(8)],
                    [16, 32, 8], [x_pad, wcat, deg0, deg1])


def _head(h, wcat, dinv):
    n_pad, F = h.shape

    def body(h_ref, w_ref, dv_ref, a_ref, y_ref):
        d = dv_ref[:, 0]
        hh = jnp.dot(h_ref[:], w_ref[:], preferred_element_type=jnp.float32)
        a_ref[:] = hh[:, :16]
        y_ref[:] = hh[:, 16:48] * d[:, None]

    return _tc_call(body, n_pad,
                    [_row_spec(F), _full_spec(F, 48), _row_spec(8)],
                    [16, 32], [h, wcat, dinv])


def _mid(v0, v1, dinv):
    n_pad, C = v0.shape

    def body(v0_ref, v1_ref, dv_ref, w_ref):
        d = dv_ref[:, 0]
        w_ref[:] = -(d * d)[:, None] * (v0_ref[:] + v1_ref[:])

    return _tc_call(body, n_pad,
                    [_row_spec(C), _row_spec(C), _row_spec(8)],
                    [C], [v0, v1, dinv])[0]


def _tail_reorder(A, v0, v1, z0, z1, dinv, b, emit_u):
    n_pad = A.shape[0]

    def body(a_ref, v0_ref, v1_ref, z0_ref, z1_ref, dv_ref, b_ref, *outs):
        d = dv_ref[:, 0]
        V1 = (v0_ref[:] + v1_ref[:])[:, :16]
        Z = z0_ref[:] + z1_ref[:]
        o = jax.nn.relu(a_ref[:] - d[:, None] * (V1 + 2.0 * Z) + b_ref[:])
        outs[0][:] = o
        if emit_u:
            outs[1][:] = o * d[:, None]

    out_cs = [16, 16] if emit_u else [16]
    return _tc_call(body, n_pad,
                    [_row_spec(16), _row_spec(32), _row_spec(32),
                     _row_spec(16), _row_spec(16), _row_spec(8),
                     _full_spec(1, 16)],
                    out_cs, [A, v0, v1, z0, z1, dinv, b])


def _combine(h, vs, zs, dinv, Ws, b, u_split):
    n_pad, Cin = h.shape
    Cout = Ws.shape[1]
    nv = len(vs)

    def body(*refs):
        h_ref = refs[0]
        v_refs = refs[1:1 + nv]
        z_refs = refs[1 + nv:1 + 2 * nv]
        dv_ref, w_ref, b_ref = refs[1 + 2 * nv:4 + 2 * nv]
        outs = refs[4 + 2 * nv:]
        d = dv_ref[:, 0]
        if nv == 2:
            v = v_refs[0][:] + v_refs[1][:]
            z = z_refs[0][:] + z_refs[1][:]
        else:
            v = jnp.concatenate([v_refs[0][:] + v_refs[1][:],
                                 v_refs[2][:] + v_refs[3][:]], axis=1)
            z = jnp.concatenate([z_refs[0][:] + z_refs[1][:],
                                 z_refs[2][:] + z_refs[3][:]], axis=1)
        cat = jnp.concatenate([h_ref[:], -d[:, None] * v,
                               -2.0 * d[:, None] * z], axis=1)
        o = jax.nn.relu(
            jnp.dot(cat, w_ref[:], preferred_element_type=jnp.float32)
            + b_ref[:])
        outs[0][:] = o
        if u_split == 1:
            outs[1][:] = o * d[:, None]
        elif u_split == 2:
            u = o * d[:, None]
            outs[1][:] = u[:, :Cout // 2]
            outs[2][:] = u[:, Cout // 2:]

    vc = Cin if nv == 2 else Cin // 2
    in_specs = ([_row_spec(Cin)] + [_row_spec(vc)] * (2 * nv)
                + [_row_spec(8), _full_spec(3 * Cin, Cout),
                   _full_spec(1, Cout)])
    out_cs = {0: [Cout], 1: [Cout, Cout], 2: [Cout, Cout // 2, Cout // 2]}[u_split]
    return _tc_call(body, n_pad, in_specs, out_cs,
                    [h] + list(vs) + list(zs) + [dinv, Ws, b])


def _wstack(W):
    return jnp.concatenate([W[0] - W[2], W[1], W[2]], axis=0)


def _wcat(W, pad_out=0):
    cols = [W[0] - W[2], W[1], W[2]]
    if pad_out:
        cols = [jnp.pad(c, ((0, 0), (0, pad_out))) for c in cols]
    return jnp.concatenate(cols, axis=1)


def kernel(x, edge_index, W1, b1, W2, b2, W3, b3, W4, b4, W5, b5):
    N, F = x.shape
    E = edge_index.shape[1]
    n_pad = ((N + BR - 1) // BR) * BR
    F_pad = ((F + 127) // 128) * 128
    ng8 = (E + NW * GB - 1) // (NW * GB)
    e_pad = NW * ng8 * GB
    ng4 = e_pad // (NW * 512)

    x_pad = jnp.pad(x, ((0, n_pad - N), (0, F_pad - F)))
    src = edge_index[0]
    dst = edge_index[1]
    fill = jnp.full((e_pad - E,), n_pad - 1, jnp.int32)
    srcf = jnp.concatenate([src, fill])
    dstf = jnp.concatenate([dst, fill])
    srcg = srcf.reshape(NW * ng8, GB // SUB, SUB)
    dstg = dstf.reshape(NW * ng8, GB // SUB, SUB)
    srcg4 = srcf.reshape(NW * ng4, 512 // SUB, SUB)
    dstg4 = dstf.reshape(NW * ng4, 512 // SUB, SUB)

    prop8 = _make_prop(n_pad, 8, ng8, GB)
    prop16 = _make_prop(n_pad, 16, ng8, GB)
    prop32 = _make_prop(n_pad, 32, ng4, 512)
    z8 = jnp.zeros((n_pad // NS, 8), jnp.float32)
    z16 = jnp.zeros((n_pad // NS, 16), jnp.float32)
    z32 = jnp.zeros((n_pad // NS, 32), jnp.float32)

    ind = jnp.zeros((n_pad, 8), jnp.float32).at[:N].set(1.0)
    degp = prop8(ind, srcg, srcg, z8)

    wc1 = jnp.pad(_wcat(W1), ((0, F_pad - F), (0, 0)))
    A1, Y1, dinv = _head_deg(x_pad, wc1, degp[0], degp[1])
    Vp = prop32(Y1, srcg4, dstg4, z32)
    G = _mid(Vp[0, :, 16:32], Vp[1, :, 16:32], dinv)
    Zp = prop16(G, srcg, dstg, z16)
    h1, u2 = _tail_reorder(A1, Vp[0], Vp[1], Zp[0], Zp[1], dinv,
                           b1.reshape(1, 16), True)

    vp = prop16(u2, srcg, dstg, z16)
    w = _mid(vp[0], vp[1], dinv)
    zp = prop16(w, srcg, dstg, z16)
    h2, u3 = _combine(h1, [vp[0], vp[1]], [zp[0], zp[1]], dinv,
                      _wstack(W2), b2.reshape(1, 32), 1)

    vp = prop32(u3, srcg4, dstg4, z32)
    w = _mid(vp[0], vp[1], dinv)
    zp = prop32(w, srcg4, dstg4, z32)
    h3, u4a, u4b = _combine(h2, [vp[0], vp[1]], [zp[0], zp[1]], dinv,
                            _wstack(W3), b3.reshape(1, 64), 2)

    vpa = prop32(u4a, srcg4, dstg4, z32)
    vpb = prop32(u4b, srcg4, dstg4, z32)
    wa = _mid(vpa[0], vpa[1], dinv)
    wb = _mid(vpb[0], vpb[1], dinv)
    zpa = prop32(wa, srcg4, dstg4, z32)
    zpb = prop32(wb, srcg4, dstg4, z32)
    (h4,) = _combine(h3, [vpa[0], vpa[1], vpb[0], vpb[1]],
                     [zpa[0], zpa[1], zpb[0], zpb[1]], dinv,
                     _wstack(W4), b4.reshape(1, 128), 0)

    wc5 = _wcat(W5, pad_out=6)
    A5, Y5 = _head(h4, wc5, dinv)
    Vp5 = prop32(Y5, srcg4, dstg4, z32)
    G5 = _mid(Vp5[0, :, 16:32], Vp5[1, :, 16:32], dinv)
    Zp5 = prop16(G5, srcg, dstg, z16)
    (h5,) = _tail_reorder(A5, Vp5[0], Vp5[1], Zp5[0], Zp5[1], dinv,
                          jnp.pad(b5, (0, 6)).reshape(1, 16), False)

    return h5[:N, :10]
